# baseline (speedup 1.0000x reference)
import functools

import jax
import jax.numpy as jnp
from jax import lax
from jax.experimental import pallas as pl
from jax.experimental.pallas import tpu as pltpu
from jax.experimental.pallas import tpu_sc as plsc

N_NODES = 10000
N_EDGES = 320000
FEAT = 128

NC = 2
NS = 16
NW = NC * NS

CHUNK = 128
TOTAL_CHUNKS = N_EDGES // CHUNK
CHUNKS_PER_TILE = TOTAL_CHUNKS // NW
EXTRA_BASE = NW * CHUNKS_PER_TILE
N_EXTRA = TOTAL_CHUNKS - EXTRA_BASE
STAGE = 26
N_STAGES = CHUNKS_PER_TILE // STAGE
STAGE_BUF = 40
AB_MAX = (TOTAL_CHUNKS - N_EXTRA - STAGE_BUF) // 8 * 8

ROWS_PER_TILE = -(-(N_NODES + 1) // (NS * 8)) * 8
AGG_ROWS = ROWS_PER_TILE * NS

MLP_BLOCK = 1000
MLP_GRID = N_NODES // MLP_BLOCK


def _sc_aggregate(x, src, dst, zeros):
    mesh = plsc.VectorSubcoreMesh(core_axis_name="c", subcore_axis_name="s")

    @functools.partial(
        pl.kernel,
        out_type=jax.ShapeDtypeStruct((NC, AGG_ROWS, FEAT), jnp.float32),
        mesh=mesh,
        scratch_types=[
            pltpu.VMEM((STAGE_BUF, CHUNK), jnp.int32),
            pltpu.VMEM((STAGE_BUF, CHUNK), jnp.int32),
            pltpu.VMEM((CHUNK, FEAT), jnp.float32),
            pltpu.VMEM((CHUNK, FEAT), jnp.float32),
            pltpu.VMEM_SHARED((AGG_ROWS, FEAT), jnp.float32),
            pltpu.SemaphoreType.DMA,
            pltpu.SemaphoreType.DMA,
        ],
    )
    def agg_kernel(x_hbm, src_hbm, dst_hbm, zeros_hbm, out_hbm,
                   src_v, dst_v, rows0_v, rows1_v, agg_sh, gsem0, gsem1):
        cid = lax.axis_index("c")
        sid = lax.axis_index("s")
        wid = cid * NS + sid
        row0 = sid * ROWS_PER_TILE

        pltpu.sync_copy(zeros_hbm.at[pl.ds(0, ROWS_PER_TILE)],
                        agg_sh.at[pl.ds(row0, ROWS_PER_TILE)])
        plsc.subcore_barrier()

        bufs = (rows0_v, rows1_v)
        gsems = (gsem0, gsem1)

        def gather(c, b):
            pltpu.async_copy(x_hbm.at[src_v.at[c]], bufs[b], gsems[b])

        def gather_wait(c, b):
            pltpu.make_async_copy(x_hbm.at[src_v.at[c]], bufs[b],
                                  gsems[b]).wait()

        def scatter(c, b):
            pltpu.sync_copy(bufs[b], agg_sh.at[dst_v.at[c]], add=True)

        for h in range(N_STAGES):
            chunk_base = wid * CHUNKS_PER_TILE + h * STAGE
            ab = jnp.minimum(chunk_base // 8 * 8, AB_MAX)
            off = chunk_base - ab
            pltpu.sync_copy(src_hbm.at[pl.ds(ab, STAGE_BUF)], src_v)
            pltpu.sync_copy(dst_hbm.at[pl.ds(ab, STAGE_BUF)], dst_v)
            gather(off, 0)

            def body(g, carry):
                c = off + 2 * g
                gather(c + 1, 1)
                gather_wait(c, 0)
                scatter(c, 0)

                @pl.when(g < STAGE // 2 - 1)
                def _():
                    gather(c + 2, 0)

                gather_wait(c + 1, 1)
                scatter(c + 1, 1)
                return carry

            lax.fori_loop(0, STAGE // 2, body, 0, unroll=False)

        @pl.when(wid < N_EXTRA)
        def _():
            pltpu.sync_copy(src_hbm.at[pl.ds(EXTRA_BASE, N_EXTRA)],
                            src_v.at[pl.ds(0, N_EXTRA)])
            pltpu.sync_copy(dst_hbm.at[pl.ds(EXTRA_BASE, N_EXTRA)],
                            dst_v.at[pl.ds(0, N_EXTRA)])
            gather(wid, 0)
            gather_wait(wid, 0)
            scatter(wid, 0)

        plsc.subcore_barrier()

        pltpu.sync_copy(agg_sh.at[pl.ds(row0, ROWS_PER_TILE)],
                        out_hbm.at[cid, pl.ds(row0, ROWS_PER_TILE)])

    return agg_kernel(x, src, dst, zeros)


def _mlp_body(eps_ref, x_ref, p_ref, w1_ref, b1_ref, w2_ref, b2_ref, y_ref):
    scale = 1.0 + eps_ref[0]
    out = scale * x_ref[...] + p_ref[0] + p_ref[1]
    h = jnp.maximum(
        jnp.dot(out, w1_ref[...], preferred_element_type=jnp.float32)
        + b1_ref[...], 0.0)
    y_ref[...] = (
        jnp.dot(h, w2_ref[...], preferred_element_type=jnp.float32)
        + b2_ref[...])


def _tc_mlp(eps, x, partials, W1, b1, W2, b2):
    return pl.pallas_call(
        _mlp_body,
        grid=(MLP_GRID,),
        in_specs=[
            pl.BlockSpec(memory_space=pltpu.SMEM),
            pl.BlockSpec((MLP_BLOCK, FEAT), lambda i: (i, 0)),
            pl.BlockSpec((NC, MLP_BLOCK, FEAT), lambda i: (0, i, 0)),
            pl.BlockSpec((FEAT, FEAT), lambda i: (0, 0)),
            pl.BlockSpec((1, FEAT), lambda i: (0, 0)),
            pl.BlockSpec((FEAT, FEAT), lambda i: (0, 0)),
            pl.BlockSpec((1, FEAT), lambda i: (0, 0)),
        ],
        out_specs=pl.BlockSpec((MLP_BLOCK, FEAT), lambda i: (i, 0)),
        out_shape=jax.ShapeDtypeStruct((N_NODES, FEAT), jnp.float32),
    )(eps, x, partials, W1, b1, W2, b2)


@jax.jit
def kernel(x, edge_index, eps, W1, b1, W2, b2):
    src_p = edge_index[0].reshape(TOTAL_CHUNKS, CHUNK)
    dst_p = edge_index[1].reshape(TOTAL_CHUNKS, CHUNK)
    zeros = jnp.zeros((ROWS_PER_TILE, FEAT), jnp.float32)

    partials = _sc_aggregate(x, src_p, dst_p, zeros)
    return _tc_mlp(eps.reshape(1), x, partials, W1,
                   b1.reshape(1, FEAT), W2, b2.reshape(1, FEAT))


# speedup vs baseline: 1.0279x; 1.0279x over previous
import functools

import jax
import jax.numpy as jnp
from jax import lax
from jax.experimental import pallas as pl
from jax.experimental.pallas import tpu as pltpu
from jax.experimental.pallas import tpu_sc as plsc

N_NODES = 10000
N_EDGES = 320000
FEAT = 128

NC = 2
NS = 16
NW = NC * NS

CHUNK = 128
CHUNKS_PER_TILE = 80
STAGE = 40
TOTAL_CHUNKS = CHUNKS_PER_TILE * NW
E_PAD = TOTAL_CHUNKS * CHUNK

ROWS_PER_TILE = -(-(N_NODES + 1) // (NS * 8)) * 8
AGG_ROWS = ROWS_PER_TILE * NS
N_TRASH = AGG_ROWS - N_NODES

MLP_BLOCK = 1000
MLP_GRID = N_NODES // MLP_BLOCK


def _sc_aggregate(x, src, dst, zeros):
    mesh = plsc.VectorSubcoreMesh(core_axis_name="c", subcore_axis_name="s")

    @functools.partial(
        pl.kernel,
        out_type=jax.ShapeDtypeStruct((NC, AGG_ROWS, FEAT), jnp.float32),
        mesh=mesh,
        scratch_types=[
            pltpu.VMEM((STAGE, CHUNK), jnp.int32),
            pltpu.VMEM((STAGE, CHUNK), jnp.int32),
            pltpu.VMEM((CHUNK, FEAT), jnp.float32),
            pltpu.VMEM((CHUNK, FEAT), jnp.float32),
            pltpu.VMEM_SHARED((AGG_ROWS, FEAT), jnp.float32),
            pltpu.SemaphoreType.DMA,
            pltpu.SemaphoreType.DMA,
        ],
    )
    def agg_kernel(x_hbm, src_hbm, dst_hbm, zeros_hbm, out_hbm,
                   src_v, dst_v, rows0_v, rows1_v, agg_sh, gsem0, gsem1):
        cid = lax.axis_index("c")
        sid = lax.axis_index("s")
        wid = cid * NS + sid
        row0 = sid * ROWS_PER_TILE

        pltpu.sync_copy(zeros_hbm.at[pl.ds(0, ROWS_PER_TILE)],
                        agg_sh.at[pl.ds(row0, ROWS_PER_TILE)])
        plsc.subcore_barrier()

        bufs = (rows0_v, rows1_v)
        gsems = (gsem0, gsem1)

        def gather(c, b):
            pltpu.async_copy(x_hbm.at[src_v.at[c]], bufs[b], gsems[b])

        def gather_wait(c, b):
            pltpu.make_async_copy(x_hbm.at[src_v.at[c]], bufs[b],
                                  gsems[b]).wait()

        def scatter(c, b):
            pltpu.sync_copy(bufs[b], agg_sh.at[dst_v.at[c]], add=True)

        for h in range(CHUNKS_PER_TILE // STAGE):
            pltpu.sync_copy(src_hbm.at[wid, pl.ds(h * STAGE, STAGE)], src_v)
            pltpu.sync_copy(dst_hbm.at[wid, pl.ds(h * STAGE, STAGE)], dst_v)
            gather(0, 0)

            def body(g, carry):
                c = 2 * g
                gather(c + 1, 1)
                gather_wait(c, 0)
                scatter(c, 0)

                @pl.when(g < STAGE // 2 - 1)
                def _():
                    gather(c + 2, 0)

                gather_wait(c + 1, 1)
                scatter(c + 1, 1)
                return carry

            lax.fori_loop(0, STAGE // 2, body, 0, unroll=False)
        plsc.subcore_barrier()

        pltpu.sync_copy(agg_sh.at[pl.ds(row0, ROWS_PER_TILE)],
                        out_hbm.at[cid, pl.ds(row0, ROWS_PER_TILE)])

    return agg_kernel(x, src, dst, zeros)


def _mlp_body(eps_ref, x_ref, p_ref, w1_ref, b1_ref, w2_ref, b2_ref, y_ref):
    scale = 1.0 + eps_ref[0]
    out = scale * x_ref[...] + p_ref[0] + p_ref[1]
    h = jnp.maximum(
        jnp.dot(out, w1_ref[...], preferred_element_type=jnp.float32)
        + b1_ref[...], 0.0)
    y_ref[...] = (
        jnp.dot(h, w2_ref[...], preferred_element_type=jnp.float32)
        + b2_ref[...])


def _tc_mlp(eps, x, partials, W1, b1, W2, b2):
    return pl.pallas_call(
        _mlp_body,
        grid=(MLP_GRID,),
        in_specs=[
            pl.BlockSpec(memory_space=pltpu.SMEM),
            pl.BlockSpec((MLP_BLOCK, FEAT), lambda i: (i, 0)),
            pl.BlockSpec((NC, MLP_BLOCK, FEAT), lambda i: (0, i, 0)),
            pl.BlockSpec((FEAT, FEAT), lambda i: (0, 0)),
            pl.BlockSpec((1, FEAT), lambda i: (0, 0)),
            pl.BlockSpec((FEAT, FEAT), lambda i: (0, 0)),
            pl.BlockSpec((1, FEAT), lambda i: (0, 0)),
        ],
        out_specs=pl.BlockSpec((MLP_BLOCK, FEAT), lambda i: (i, 0)),
        out_shape=jax.ShapeDtypeStruct((N_NODES, FEAT), jnp.float32),
    )(eps, x, partials, W1, b1, W2, b2)


@jax.jit
def kernel(x, edge_index, eps, W1, b1, W2, b2):
    src = edge_index[0]
    dst = edge_index[1]
    pad = E_PAD - N_EDGES
    pad_src = jnp.arange(pad, dtype=jnp.int32) % N_NODES
    src_p = jnp.concatenate([src, pad_src]).reshape(NW, CHUNKS_PER_TILE, CHUNK)
    trash = N_NODES + jnp.arange(pad, dtype=jnp.int32) % N_TRASH
    dst_p = jnp.concatenate([dst, trash]).reshape(NW, CHUNKS_PER_TILE, CHUNK)
    zeros = jnp.zeros((ROWS_PER_TILE, FEAT), jnp.float32)

    partials = _sc_aggregate(x, src_p, dst_p, zeros)
    return _tc_mlp(eps.reshape(1), x, partials, W1,
                   b1.reshape(1, FEAT), W2, b2.reshape(1, FEAT))


# speedup vs baseline: 1.0513x; 1.0228x over previous
import functools

import jax
import jax.numpy as jnp
from jax import lax
from jax.experimental import pallas as pl
from jax.experimental.pallas import tpu as pltpu
from jax.experimental.pallas import tpu_sc as plsc

N_NODES = 10000
N_EDGES = 320000
FEAT = 128

NC = 2
NS = 16
NW = NC * NS

CHUNK = 128
CHUNKS_PER_TILE = 80
STAGE = 40
TOTAL_CHUNKS = CHUNKS_PER_TILE * NW
E_PAD = TOTAL_CHUNKS * CHUNK

ROWS_PER_TILE = -(-(N_NODES + 1) // (NS * 8)) * 8
AGG_ROWS = ROWS_PER_TILE * NS
N_TRASH = AGG_ROWS - N_NODES

MLP_BLOCK = 2000
MLP_GRID = N_NODES // MLP_BLOCK


def _sc_aggregate(x, src, dst, zeros):
    mesh = plsc.VectorSubcoreMesh(core_axis_name="c", subcore_axis_name="s")

    @functools.partial(
        pl.kernel,
        out_type=jax.ShapeDtypeStruct((NC, AGG_ROWS, FEAT), jnp.float32),
        mesh=mesh,
        scratch_types=[
            pltpu.VMEM((STAGE, CHUNK), jnp.int32),
            pltpu.VMEM((STAGE, CHUNK), jnp.int32),
            pltpu.VMEM((CHUNK, FEAT), jnp.float32),
            pltpu.VMEM((CHUNK, FEAT), jnp.float32),
            pltpu.VMEM_SHARED((AGG_ROWS, FEAT), jnp.float32),
            pltpu.SemaphoreType.DMA,
            pltpu.SemaphoreType.DMA,
        ],
    )
    def agg_kernel(x_hbm, src_hbm, dst_hbm, zeros_hbm, out_hbm,
                   src_v, dst_v, rows0_v, rows1_v, agg_sh, gsem0, gsem1):
        cid = lax.axis_index("c")
        sid = lax.axis_index("s")
        wid = cid * NS + sid
        row0 = sid * ROWS_PER_TILE

        pltpu.sync_copy(zeros_hbm.at[pl.ds(0, ROWS_PER_TILE)],
                        agg_sh.at[pl.ds(row0, ROWS_PER_TILE)])
        plsc.subcore_barrier()

        bufs = (rows0_v, rows1_v)
        gsems = (gsem0, gsem1)

        def gather(c, b):
            pltpu.async_copy(x_hbm.at[src_v.at[c]], bufs[b], gsems[b])

        def gather_wait(c, b):
            pltpu.make_async_copy(x_hbm.at[src_v.at[c]], bufs[b],
                                  gsems[b]).wait()

        def scatter(c, b):
            pltpu.sync_copy(bufs[b], agg_sh.at[dst_v.at[c]], add=True)

        for h in range(CHUNKS_PER_TILE // STAGE):
            pltpu.sync_copy(src_hbm.at[wid, pl.ds(h * STAGE, STAGE)], src_v)
            pltpu.sync_copy(dst_hbm.at[wid, pl.ds(h * STAGE, STAGE)], dst_v)
            gather(0, 0)

            def body(g, carry):
                c = 2 * g
                gather(c + 1, 1)
                gather_wait(c, 0)
                scatter(c, 0)

                @pl.when(g < STAGE // 2 - 1)
                def _():
                    gather(c + 2, 0)

                gather_wait(c + 1, 1)
                scatter(c + 1, 1)
                return carry

            lax.fori_loop(0, STAGE // 2, body, 0, unroll=False)
        plsc.subcore_barrier()

        pltpu.sync_copy(agg_sh.at[pl.ds(row0, ROWS_PER_TILE)],
                        out_hbm.at[cid, pl.ds(row0, ROWS_PER_TILE)])

    return agg_kernel(x, src, dst, zeros)


def _mlp_body(eps_ref, x_ref, p_ref, w1_ref, b1_ref, w2_ref, b2_ref, y_ref):
    scale = 1.0 + eps_ref[0]
    out = scale * x_ref[...] + p_ref[0] + p_ref[1]
    h = jnp.maximum(
        jnp.dot(out, w1_ref[...], preferred_element_type=jnp.float32)
        + b1_ref[...], 0.0)
    y_ref[...] = (
        jnp.dot(h, w2_ref[...], preferred_element_type=jnp.float32)
        + b2_ref[...])


def _tc_mlp(eps, x, partials, W1, b1, W2, b2):
    return pl.pallas_call(
        _mlp_body,
        grid=(MLP_GRID,),
        in_specs=[
            pl.BlockSpec(memory_space=pltpu.SMEM),
            pl.BlockSpec((MLP_BLOCK, FEAT), lambda i: (i, 0)),
            pl.BlockSpec((NC, MLP_BLOCK, FEAT), lambda i: (0, i, 0)),
            pl.BlockSpec((FEAT, FEAT), lambda i: (0, 0)),
            pl.BlockSpec((1, FEAT), lambda i: (0, 0)),
            pl.BlockSpec((FEAT, FEAT), lambda i: (0, 0)),
            pl.BlockSpec((1, FEAT), lambda i: (0, 0)),
        ],
        out_specs=pl.BlockSpec((MLP_BLOCK, FEAT), lambda i: (i, 0)),
        out_shape=jax.ShapeDtypeStruct((N_NODES, FEAT), jnp.float32),
    )(eps, x, partials, W1, b1, W2, b2)


@jax.jit
def kernel(x, edge_index, eps, W1, b1, W2, b2):
    src = edge_index[0]
    dst = edge_index[1]
    pad = E_PAD - N_EDGES
    pad_src = jnp.arange(pad, dtype=jnp.int32) % N_NODES
    src_p = jnp.concatenate([src, pad_src]).reshape(NW, CHUNKS_PER_TILE, CHUNK)
    trash = N_NODES + jnp.arange(pad, dtype=jnp.int32) % N_TRASH
    dst_p = jnp.concatenate([dst, trash]).reshape(NW, CHUNKS_PER_TILE, CHUNK)
    zeros = jnp.zeros((ROWS_PER_TILE, FEAT), jnp.float32)

    partials = _sc_aggregate(x, src_p, dst_p, zeros)
    return _tc_mlp(eps.reshape(1), x, partials, W1,
                   b1.reshape(1, FEAT), W2, b2.reshape(1, FEAT))
